# baseline (speedup 1.0000x reference)
import functools

import jax
import jax.numpy as jnp
from jax import lax
from jax.experimental import pallas as pl
from jax.experimental.pallas import tpu as pltpu

B, N, M = 4, 4096, 4096
MS = 3584
MV = M - MS


def _split3(v):
    h1 = v.astype(jnp.bfloat16)
    r1 = v - h1.astype(jnp.float32)
    h2 = r1.astype(jnp.bfloat16)
    h3 = (r1 - h2.astype(jnp.float32)).astype(jnp.bfloat16)
    return h1, h2, h3


def _chamfer_body(x1_ref, y1_ref, x2_ref, y2_ref, out_ref):
    b = pl.program_id(0)

    x1 = x1_ref[0, 0, :].reshape(N, 1)
    y1 = y1_ref[0, 0, :].reshape(N, 1)
    x2 = x2_ref[0, 0, :].reshape(1, M)
    y2 = y2_ref[0, 0, :].reshape(1, M)

    a2 = x1 * x1 + y1 * y1
    a2h1, a2h2, a2h3 = _split3(a2)
    ones_c = jnp.ones((N, 1), jnp.bfloat16)
    am = jnp.concatenate(
        [
            (x1.astype(jnp.bfloat16) * jnp.bfloat16(-2.0)),
            (y1.astype(jnp.bfloat16) * jnp.bfloat16(-2.0)),
            ones_c,
            ones_c,
            ones_c,
            a2h1,
            a2h2,
            a2h3,
        ],
        axis=1,
    )

    x2m = x2[:, :MS]
    y2m = y2[:, :MS]
    b2 = x2m * x2m + y2m * y2m
    b2h1, b2h2, b2h3 = _split3(b2)
    ones_r = jnp.ones((1, MS), jnp.bfloat16)
    bm = jnp.concatenate(
        [
            x2m.astype(jnp.bfloat16),
            y2m.astype(jnp.bfloat16),
            b2h1,
            b2h2,
            b2h3,
            ones_r,
            ones_r,
            ones_r,
        ],
        axis=0,
    )

    s = lax.dot_general(
        am, bm, (((1,), (0,)), ((), ())),
        preferred_element_type=jnp.float32,
    )

    x2v = x2[:, MS:]
    y2v = y2[:, MS:]
    bx1 = x1.astype(jnp.bfloat16).astype(jnp.float32)
    by1 = y1.astype(jnp.bfloat16).astype(jnp.float32)
    bx2 = x2v.astype(jnp.bfloat16).astype(jnp.float32)
    by2 = y2v.astype(jnp.bfloat16).astype(jnp.float32)
    ab = bx1 * bx2 + by1 * by2
    b2v = x2v * x2v + y2v * y2v
    sv = (a2 + b2v) - (ab + ab)

    rowmin = jnp.minimum(jnp.min(s, axis=1), jnp.min(sv, axis=1))
    colsum = jnp.sum(jnp.maximum(jnp.min(s, axis=0), 0.0)) + jnp.sum(
        jnp.maximum(jnp.min(sv, axis=0), 0.0)
    )

    cost = (
        jnp.sum(jnp.maximum(rowmin, 0.0)) * (1.0 / N)
        + colsum * (1.0 / M)
    )

    @pl.when(b == 0)
    def _init():
        out_ref[0, 0] = cost

    @pl.when(b != 0)
    def _acc():
        out_ref[0, 0] += cost


@jax.jit
def kernel(points1, points2):
    x1 = points1[..., 0].reshape(B, 1, N)
    y1 = points1[..., 1].reshape(B, 1, N)
    x2 = points2[..., 0].reshape(B, 1, M)
    y2 = points2[..., 1].reshape(B, 1, M)

    out = pl.pallas_call(
        _chamfer_body,
        grid=(B,),
        in_specs=[
            pl.BlockSpec((1, 1, N), lambda b: (b, 0, 0)),
            pl.BlockSpec((1, 1, N), lambda b: (b, 0, 0)),
            pl.BlockSpec((1, 1, M), lambda b: (b, 0, 0)),
            pl.BlockSpec((1, 1, M), lambda b: (b, 0, 0)),
        ],
        out_specs=pl.BlockSpec(
            (1, 1), lambda b: (0, 0), memory_space=pltpu.SMEM
        ),
        out_shape=jax.ShapeDtypeStruct((1, 1), jnp.float32),
    )(x1, y1, x2, y2)
    return out[0, 0]
